# baseline (speedup 1.0000x reference)
import functools

import jax
import jax.numpy as jnp
from jax import lax
from jax.experimental import pallas as pl
from jax.experimental.pallas import tpu as pltpu
from jax.experimental.pallas import tpu_sc as plsc

_CH = 80000
_ACC = _CH + 16


@functools.lru_cache(maxsize=None)
def _make_sc_scatter(KP, TR, OR, C):
    NT = 16
    PT = KP // NT
    NSB = PT // 1024
    mesh = plsc.VectorSubcoreMesh(core_axis_name="c", subcore_axis_name="s")
    ZR = _ACC // NT
    DR = _CH // NT
    NZF = ZR // 128
    ZREM = ZR - NZF * 128

    @functools.partial(
        pl.kernel, mesh=mesh,
        out_type=jax.ShapeDtypeStruct((8, OR, 16), jnp.float32),
        compiler_params=pltpu.CompilerParams(use_tc_tiling_on_sc=False),
        scratch_types=[
            pltpu.VMEM_SHARED((_ACC, 16), jnp.float32),
            pltpu.VMEM((128, 16), jnp.float32),
            pltpu.VMEM((8, 128), jnp.int32),
            pltpu.VMEM((8, 128), jnp.int32),
            pltpu.VMEM((8, 128), jnp.int32),
            pltpu.VMEM((8, 128), jnp.int32),
            pltpu.VMEM((1024, 16), jnp.float32),
            pltpu.VMEM((1024, 16), jnp.float32),
            pltpu.SemaphoreType.DMA,
            pltpu.SemaphoreType.DMA,
            pltpu.SemaphoreType.DMA,
            pltpu.SemaphoreType.DMA,
        ])
    def k(table_h, gs_h, dl_h, zrows_h, out_h,
          acc, zbuf, gb0, gb1, db0, db1, rb0, rb1,
          sem_g0, sem_g1, sem_s, sem_z):
        core = lax.axis_index("c")
        tid = lax.axis_index("s")
        pltpu.sync_copy(zrows_h, zbuf)
        for ci in range(C // 2):
            cidx = core * (C // 2) + ci
            base = cidx * _CH

            def slice_body(s, _):
                z0 = tid * ZR
                zd = [pltpu.async_copy(
                    zbuf, acc.at[pl.ds(z0 + q * 128, 128)], sem_z)
                    for q in range(NZF)]
                if ZREM:
                    zd.append(pltpu.async_copy(
                        zbuf.at[pl.ds(0, ZREM)],
                        acc.at[pl.ds(z0 + NZF * 128, ZREM)], sem_z))
                for d in zd:
                    d.wait()
                plsc.subcore_barrier()

                def m_body(m, _):
                    j0 = 2 * m
                    sb0 = tid * NSB + j0
                    pltpu.sync_copy(gs_h.at[s, sb0], gb0)
                    pltpu.sync_copy(dl_h.at[cidx, sb0], db0)
                    g0 = [pltpu.async_copy(
                        table_h.at[gb0.at[b]],
                        rb0.at[pl.ds(b * 128, 128)], sem_g0)
                        for b in range(8)]
                    pltpu.sync_copy(gs_h.at[s, sb0 + 1], gb1)
                    pltpu.sync_copy(dl_h.at[cidx, sb0 + 1], db1)
                    g1 = [pltpu.async_copy(
                        table_h.at[gb1.at[b]],
                        rb1.at[pl.ds(b * 128, 128)], sem_g1)
                        for b in range(8)]
                    sd = []
                    for b in range(8):
                        g0[b].wait()
                        sd.append(pltpu.async_copy(
                            rb0.at[pl.ds(b * 128, 128)],
                            acc.at[db0.at[b]], sem_s, add=True))
                    for b in range(8):
                        g1[b].wait()
                        sd.append(pltpu.async_copy(
                            rb1.at[pl.ds(b * 128, 128)],
                            acc.at[db1.at[b]], sem_s, add=True))
                    for d in sd:
                        d.wait()
                    return _

                lax.fori_loop(0, NSB // 2, m_body, None)
                plsc.subcore_barrier()
                d0 = tid * DR
                pltpu.sync_copy(acc.at[pl.ds(d0, DR)],
                                out_h.at[s, pl.ds(base + d0, DR)])
                plsc.subcore_barrier()
                return _

            lax.fori_loop(0, 8, slice_body, None)

    return k


def _sc_scatter_rows(table2d, gs_all, dl_all):
    C = dl_all.shape[0]
    KP = gs_all.shape[1] * 1024
    OR = C * _CH
    zrows = jnp.zeros((128, 16), jnp.float32)
    k = _make_sc_scatter(KP, table2d.shape[0], OR, C)
    return k(table2d, gs_all, dl_all, zrows)


def _sc_make_idx(gidx8, gdst, C):
    K = gidx8.shape[0]
    KP = -(-K // 16384) * 16384
    gidx8 = jnp.pad(gidx8, (0, KP - K))
    gdst = jnp.pad(gdst, (0, KP - K), constant_values=-1)
    gs = (gidx8[None, :] + jnp.arange(8, dtype=jnp.int32)[:, None])
    gs = gs.reshape(8, KP // 1024, 8, 128)
    dl = gdst[None, :] - (jnp.arange(C, dtype=jnp.int32) * _CH)[:, None]
    dl = jnp.where((dl >= 0) & (dl < _CH), dl, _CH)
    dl = dl.reshape(C, KP // 1024, 8, 128)
    return gs, dl


def _seg_sum(vals, seg, num):
    return jax.ops.segment_sum(vals, seg, num_segments=num)


def _combine_body(a_ref, b_ref, o_ref):
    o_ref[...] = a_ref[...] + b_ref[...]


def _pl_add(a, b):
    E, F = a.shape
    blk = 2000
    return pl.pallas_call(
        _combine_body,
        out_shape=jax.ShapeDtypeStruct((E, F), jnp.float32),
        grid=(E // blk,),
        in_specs=[pl.BlockSpec((blk, F), lambda i: (i, 0)),
                  pl.BlockSpec((blk, F), lambda i: (i, 0))],
        out_specs=pl.BlockSpec((blk, F), lambda i: (i, 0)),
    )(a, b)


def kernel(x, edge_index, edge_attr, line_graph_edge_index, edge_index_batch, params):
    p = params
    N, F = x.shape
    E = edge_index.shape[1]
    B = 128
    T = 3
    lg0 = line_graph_edge_index[0]
    lg1 = line_graph_edge_index[1]
    ei0, ei1 = edge_index[0], edge_index[1]
    batch = edge_index_batch

    edge_u = x @ p['Wu']
    edge_v = x @ p['Wv']
    edge_uv = edge_attr @ p['We']
    ea = (edge_u[ei0] + edge_v[ei1] + edge_uv) / 3.0

    indeg = _seg_sum(jnp.ones((lg1.shape[0],), jnp.float32), lg1, E)
    dis = (indeg + 1.0) ** -0.5

    vs2 = p['gat_W'] @ p['gat_att_src']
    vd2 = p['gat_W'] @ p['gat_att_dst']

    agg_gs, agg_dl = _sc_make_idx(lg0 * 8, lg1, 2)

    out = ea
    out_list = []
    gout_list = []
    for _ in range(T):
        aggs = _sc_scatter_rows(out.reshape(E * 8, 16), agg_gs, agg_dl)
        agg = aggs.transpose(1, 0, 2).reshape(E, F)
        out = _pl_add(ea, agg)

        h = out @ p['gat_W']
        a_s = out @ vs2
        a_d = out @ vd2
        h1 = (out @ p['att_gcn_W'])[:, 0]
        score_f = out @ p['fbtl_W'] + p['fbtl_b']

        gh1 = dis * h1
        s_lg = _seg_sum(gh1[lg0], lg1, E)
        score_s = dis * s_lg + dis * dis * h1 + p['att_gcn_b'][0]
        score = score_s[:, None] * 0.6 + score_f * 0.4

        u = a_s[lg0] + a_d[lg1]
        pos = u >= 0.0
        cval = jnp.where(pos, jnp.exp(a_s[lg0]), jnp.exp(0.2 * a_s[lg0]))
        cpos = _seg_sum(jnp.where(pos, cval, 0.0), lg1, E)
        cneg = _seg_sum(jnp.where(pos, 0.0, cval), lg1, E)
        hA = jnp.exp(a_s)[:, None] * h
        hB = jnp.exp(0.2 * a_s)[:, None] * h
        rows = jnp.where(pos[:, None], hA[lg0], hB[lg0])
        Spos = _seg_sum(jnp.where(pos[:, None], rows, 0.0), lg1, E)
        Sneg = _seg_sum(jnp.where(pos[:, None], 0.0, rows), lg1, E)
        e_self = jnp.exp(jax.nn.leaky_relu(a_s + a_d, 0.2))
        ead = jnp.exp(a_d)
        ead2 = jnp.exp(0.2 * a_d)
        z = ead * cpos + ead2 * cneg + e_self
        num = ead[:, None] * Spos + ead2[:, None] * Sneg + e_self[:, None] * h
        xf = num / (z + 1e-16)[:, None] + p['gat_b']

        es = jnp.exp(score)
        zb = _seg_sum(es, batch, B)
        scores = es / (zb[batch] + 1e-16)
        gout = _seg_sum(xf * scores, batch, B)

        out_list.append(out)
        gout_list.append(jnp.tanh(gout @ p['lin_gout_W'] + p['lin_gout_b']))

    gout_all = jnp.stack(gout_list, axis=-1)
    out_all = jnp.stack(out_list, axis=-1)
    ws = (gout_all * p['a']).sum(1, keepdims=True) + p['a_bias']
    ws = jax.nn.softmax(ws, axis=-1)
    we = ws[batch, 0, :]
    o = (out_all * we[:, None, :]).sum(-1)
    x2 = x + _seg_sum(o, ei1, N)

    def bn(v, g, b):
        return g * (v - v.mean(0)) / jnp.sqrt(v.var(0) + 1e-5) + b

    def prelu(v, w):
        return jnp.where(v >= 0.0, v, w * v)

    y = bn(x2, p['bn1_g'], p['bn1_b']) @ p['l1_W'] + p['l1_b']
    hh = prelu(bn(y, p['bn2_g'], p['bn2_b']), p['pr2']) @ p['l2_W'] + p['l2_b']
    hh = prelu(bn(hh, p['bn3_g'], p['bn3_b']), p['pr3']) @ p['l3_W'] + p['l3_b']
    y = (hh + y) / 2.0
    hh = prelu(bn(y, p['bn4_g'], p['bn4_b']), p['pr4']) @ p['l4_W'] + p['l4_b']
    y = (hh + y) / 2.0
    y = prelu(bn(y, p['bn5_g'], p['bn5_b']), p['pr5']) @ p['l5_W'] + p['l5_b']
    return y


# speedup vs baseline: 1.1835x; 1.1835x over previous
import functools

import jax
import jax.numpy as jnp
from jax import lax
from jax.experimental import pallas as pl
from jax.experimental.pallas import tpu as pltpu
from jax.experimental.pallas import tpu_sc as plsc


@functools.lru_cache(maxsize=None)
def _make_sc_bucket_scatter(KPP, TR, ROWW, NLOC16, NBUCK, merged, EE, BS=128):
    NT = 16
    NR = KPP // BS
    ACCR = NLOC16 + 16
    ZR = ACCR // NT
    DR = NLOC16 // NT
    NZF = ZR // 16
    ZREM = ZR - NZF * 16
    mesh = plsc.VectorSubcoreMesh(core_axis_name="c", subcore_axis_name="s")

    scratch = [
        pltpu.VMEM_SHARED((ACCR, ROWW), jnp.float32),
        pltpu.VMEM((48,), jnp.int32),
        pltpu.VMEM((16, ROWW), jnp.float32),
        pltpu.VMEM((BS,), jnp.int32),
        pltpu.VMEM((BS,), jnp.int32),
        pltpu.VMEM((BS,), jnp.int32),
        pltpu.VMEM((BS,), jnp.int32),
        pltpu.VMEM((BS, ROWW), jnp.float32),
        pltpu.VMEM((BS, ROWW), jnp.float32),
        pltpu.SemaphoreType.DMA,
        pltpu.SemaphoreType.DMA,
        pltpu.SemaphoreType.DMA,
        pltpu.SemaphoreType.DMA,
    ]
    if merged:
        scratch += [
            pltpu.VMEM((BS,), jnp.int32),
            pltpu.VMEM((BS,), jnp.int32),
            pltpu.VMEM((BS, 16), jnp.float32),
            pltpu.VMEM((BS, 16), jnp.float32),
        ]

    def body(*refs):
        if merged:
            (table_h, gidx_h, dloc_h, poff_h, aspack_h, lg1_h, zrows_h, out_h,
             acc, poffv, zbuf, gb0, gb1, db0, db1, rb0, rb1,
             sem_g0, sem_g1, sem_s, sem_z,
             l1v0, l1v1, rA, rB) = refs
        else:
            (table_h, gidx_h, dloc_h, poff_h, zrows_h, out_h,
             acc, poffv, zbuf, gb0, gb1, db0, db1, rb0, rb1,
             sem_g0, sem_g1, sem_s, sem_z) = refs
        core = lax.axis_index("c")
        tid = lax.axis_index("s")
        pltpu.sync_copy(zrows_h, zbuf)
        pltpu.sync_copy(poff_h, poffv)
        iot = lax.iota(jnp.int32, 16)
        zero16 = iot * 0

        def stage_and_gather(r, gb, db, l1v, rA, rB, semg):
            if merged:
                pltpu.sync_copy(gidx_h.at[r], gb)
                pltpu.sync_copy(lg1_h.at[r], l1v)
                pltpu.sync_copy(dloc_h.at[r], db)
                gA = pltpu.async_copy(aspack_h.at[gb], rA, semg)
                gB = pltpu.async_copy(aspack_h.at[l1v], rB, semg)
                gA.wait()
                gB.wait()
                for c in range(BS // 16):
                    sl = pl.ds(c * 16, 16)
                    asv = plsc.load_gather(rA, [iot + c * 16, zero16])
                    adv = plsc.load_gather(rB, [iot + c * 16, zero16 + 1])
                    neg = jnp.where(asv + adv < 0.0, 1, 0)
                    gb[sl] = gb[sl] + neg * EE
                    db[sl] = jnp.minimum(db[sl] + neg, NLOC16)
            else:
                pltpu.sync_copy(gidx_h.at[r], gb)
                pltpu.sync_copy(dloc_h.at[r], db)
            return pltpu.async_copy(table_h.at[gb], rb0 if gb is gb0 else rb1,
                                    semg)

        for ci in range(NBUCK // 2):
            b = core * (NBUCK // 2) + ci
            pv = poffv[pl.ds(b, 16)]
            p0 = pv[0]
            p1 = pv[1]
            base_r = p0 // BS
            nb = (p1 - p0) // BS // NT
            my0 = base_r + tid * nb
            z0 = tid * ZR
            zd = [pltpu.async_copy(zbuf, acc.at[pl.ds(z0 + q * 16, 16)], sem_z)
                  for q in range(NZF)]
            if ZREM:
                zd.append(pltpu.async_copy(
                    zbuf.at[pl.ds(0, ZREM)],
                    acc.at[pl.ds(z0 + NZF * 16, ZREM)], sem_z))
            for d in zd:
                d.wait()
            plsc.subcore_barrier()

            def m_body(m, _):
                r0 = my0 + 2 * m
                r1 = jnp.where(2 * m + 1 < nb, r0 + 1, NR - 1)
                g0 = stage_and_gather(r0, gb0, db0,
                                      *( (l1v0, rA, rB) if merged
                                         else (None, None, None)), sem_g0)
                g1 = stage_and_gather(r1, gb1, db1,
                                      *( (l1v1, rA, rB) if merged
                                         else (None, None, None)), sem_g1)
                g0.wait()
                s0 = pltpu.async_copy(rb0, acc.at[db0], sem_s, add=True)
                g1.wait()
                s1 = pltpu.async_copy(rb1, acc.at[db1], sem_s, add=True)
                s0.wait()
                s1.wait()
                return _

            lax.fori_loop(0, (nb + 1) // 2, m_body, None)
            plsc.subcore_barrier()
            d0 = tid * DR
            pltpu.sync_copy(acc.at[pl.ds(d0, DR)],
                            out_h.at[b, pl.ds(d0, DR)])
            plsc.subcore_barrier()

    return functools.partial(
        pl.kernel, mesh=mesh,
        out_type=jax.ShapeDtypeStruct((NBUCK, NLOC16, ROWW), jnp.float32),
        compiler_params=pltpu.CompilerParams(use_tc_tiling_on_sc=False, needs_layout_passes=False),
        scratch_types=scratch)(body)


@functools.lru_cache(maxsize=None)
def _make_sc_gather(KPC, TR, ROWW):
    NT = 16
    PT = KPC // 32
    NB = PT // 128
    mesh = plsc.VectorSubcoreMesh(core_axis_name="c", subcore_axis_name="s")

    @functools.partial(
        pl.kernel, mesh=mesh,
        out_type=jax.ShapeDtypeStruct((KPC, ROWW), jnp.float32),
        compiler_params=pltpu.CompilerParams(use_tc_tiling_on_sc=False, needs_layout_passes=False),
        scratch_types=[
            pltpu.VMEM((128,), jnp.int32),
            pltpu.VMEM((128,), jnp.int32),
            pltpu.VMEM((128, ROWW), jnp.float32),
            pltpu.VMEM((128, ROWW), jnp.float32),
            pltpu.SemaphoreType.DMA,
            pltpu.SemaphoreType.DMA,
            pltpu.SemaphoreType.DMA,
        ])
    def k(table_h, gidx_h, out_h, gb0, gb1, rb0, rb1, sem0, sem1, sem_s):
        core = lax.axis_index("c")
        tid = lax.axis_index("s")
        w = core * 16 + tid
        r00 = (w * PT) // 128

        def m_body(m, _):
            r0 = r00 + 2 * m
            pltpu.sync_copy(gidx_h.at[r0], gb0)
            g0 = pltpu.async_copy(table_h.at[gb0], rb0, sem0)
            pltpu.sync_copy(gidx_h.at[r0 + 1], gb1)
            g1 = pltpu.async_copy(table_h.at[gb1], rb1, sem1)
            g0.wait()
            s0 = pltpu.async_copy(rb0, out_h.at[pl.ds(r0 * 128, 128)], sem_s)
            g1.wait()
            s1 = pltpu.async_copy(rb1, out_h.at[pl.ds((r0 + 1) * 128, 128)],
                                  sem_s)
            s0.wait()
            s1.wait()
            return _

        lax.fori_loop(0, NB // 2, m_body, None)

    return k


def _bucketize(dst, BCH, NBUCK, payloads, trash_vals, NLOC16):
    K = dst.shape[0]
    KPP = -(-(K + NBUCK * 2048 + 2048) // 4096) * 4096
    order = jnp.argsort(dst)
    ds_ = dst[order]
    bid = ds_ // BCH
    cnt = jnp.bincount(bid, length=NBUCK)
    pc = -(-cnt // 2048) * 2048
    poff = jnp.concatenate([jnp.zeros((1,), jnp.int32),
                            jnp.cumsum(pc).astype(jnp.int32)])
    off_raw = jnp.concatenate([jnp.zeros((1,), jnp.int32),
                               jnp.cumsum(cnt).astype(jnp.int32)])
    pos = poff[bid] + (jnp.arange(K, dtype=jnp.int32) - off_raw[bid])
    outs = []
    for arr, tv in zip(payloads, trash_vals):
        base = jnp.full((KPP,), tv, jnp.int32)
        outs.append(base.at[pos].set(arr[order]))
    dloc = jnp.full((KPP,), NLOC16, jnp.int32)
    dloc = dloc.at[pos].set(ds_ % BCH)
    poff40 = jnp.zeros((48,), jnp.int32).at[:NBUCK + 1].set(poff)
    return outs, dloc, poff40, KPP


def _seg_sum(vals, seg, num):
    return jax.ops.segment_sum(vals, seg, num_segments=num)


def _combine_body(a_ref, b_ref, o_ref):
    o_ref[...] = a_ref[...] + b_ref[...]


def _pl_add(a, b):
    E, F = a.shape
    blk = 2000
    return pl.pallas_call(
        _combine_body,
        out_shape=jax.ShapeDtypeStruct((E, F), jnp.float32),
        grid=(E // blk,),
        in_specs=[pl.BlockSpec((blk, F), lambda i: (i, 0)),
                  pl.BlockSpec((blk, F), lambda i: (i, 0))],
        out_specs=pl.BlockSpec((blk, F), lambda i: (i, 0)),
    )(a, b)


def kernel(x, edge_index, edge_attr, line_graph_edge_index, edge_index_batch, params):
    p = params
    N, F = x.shape
    E = edge_index.shape[1]
    B = 128
    T = 3
    lg0 = line_graph_edge_index[0]
    lg1 = line_graph_edge_index[1]
    ei0, ei1 = edge_index[0], edge_index[1]
    batch = edge_index_batch

    z128 = jnp.zeros((16, 128), jnp.float32)
    z144 = jnp.zeros((16, 144), jnp.float32)

    (lg0p, lg1p), dloc5, poff33, KPP = _bucketize(
        lg1, 5000, 32, [lg0, lg1], [0, 0], 10000)
    dloc5x2 = jnp.minimum(dloc5 * 2, 10000)
    dloc10 = jnp.where(dloc5 == 10000, 10000, lg1p % 10000)
    poff17 = jnp.zeros((48,), jnp.int32).at[:17].set(poff33[0:34:2][:17])

    (gx2,), dlx2, poff3, KPP2 = _bucketize(
        ei1, 5000, 2, [jnp.arange(E, dtype=jnp.int32)], [0], 5008)

    edge_u = x @ p['Wu']
    edge_v = x @ p['Wv']
    edge_uv = edge_attr @ p['We']
    uv_tab = jnp.concatenate([edge_u, edge_v], axis=0)
    KPC = 327680
    gidx_uv = jnp.concatenate([ei0, ei1 + N]).astype(jnp.int32)
    gidx_uv = jnp.pad(gidx_uv, (0, KPC - 2 * E)).reshape(KPC // 128, 128)
    uvg = _make_sc_gather(KPC, 2 * N, F)(uv_tab, gidx_uv)
    ea = (uvg[:E] + uvg[E:2 * E] + edge_uv) / 3.0

    indeg = _seg_sum(jnp.ones((lg1.shape[0],), jnp.float32), lg1, E)
    dis = (indeg + 1.0) ** -0.5

    vs2 = p['gat_W'] @ p['gat_att_src']
    vd2 = p['gat_W'] @ p['gat_att_dst']

    gs2 = lg0p.reshape(KPP // 128, 128)
    dl10 = dloc10.reshape(KPP // 128, 128)
    gs64 = lg0p.reshape(KPP // 64, 64)
    dl5_64 = dloc5x2.reshape(KPP // 64, 64)
    lg1r64 = lg1p.reshape(KPP // 64, 64)

    out = ea
    out_list = []
    gout_list = []
    for _ in range(T):
        agg = _make_sc_bucket_scatter(KPP, E, 128, 10000, 16, False, E)(
            out, gs2, dl10, poff17, z128).reshape(E, F)
        out = _pl_add(ea, agg)

        h = out @ p['gat_W']
        a_s = out @ vs2
        a_d = out @ vd2
        h1 = (out @ p['att_gcn_W'])[:, 0]
        score_f = out @ p['fbtl_W'] + p['fbtl_b']
        gh1 = dis * h1

        eas = jnp.exp(a_s)
        eas2 = jnp.exp(0.2 * a_s)
        pad14 = jnp.zeros((E, 14), jnp.float32)
        TH = jnp.concatenate([
            jnp.concatenate([eas[:, None] * h, eas[:, None], gh1[:, None],
                             pad14], axis=1),
            jnp.concatenate([eas2[:, None] * h, eas2[:, None], gh1[:, None],
                             pad14], axis=1)], axis=0)
        aspack = jnp.concatenate(
            [a_s[:, None], a_d[:, None], jnp.zeros((E, 14), jnp.float32)],
            axis=1)
        gat = _make_sc_bucket_scatter(KPP, 2 * E, 144, 10000, 32, True, E,
                                      BS=64)(
            TH, gs64, dl5_64, poff33, aspack, lg1r64, z144)
        gat = gat.reshape(E, 2, 144)
        Spos = gat[:, 0, :F]
        Sneg = gat[:, 1, :F]
        cpos = gat[:, 0, F]
        cneg = gat[:, 1, F]
        s_lg = gat[:, 0, F + 1] + gat[:, 1, F + 1]

        score_s = dis * s_lg + dis * dis * h1 + p['att_gcn_b'][0]
        score = score_s[:, None] * 0.6 + score_f * 0.4

        e_self = jnp.exp(jax.nn.leaky_relu(a_s + a_d, 0.2))
        ead = jnp.exp(a_d)
        ead2 = jnp.exp(0.2 * a_d)
        z = ead * cpos + ead2 * cneg + e_self
        num = ead[:, None] * Spos + ead2[:, None] * Sneg + e_self[:, None] * h
        xf = num / (z + 1e-16)[:, None] + p['gat_b']

        es = jnp.exp(score)
        zb = _seg_sum(es, batch, B)
        scores = es / (zb[batch] + 1e-16)
        gout = _seg_sum(xf * scores, batch, B)

        out_list.append(out)
        gout_list.append(jnp.tanh(gout @ p['lin_gout_W'] + p['lin_gout_b']))

    gout_all = jnp.stack(gout_list, axis=-1)
    out_all = jnp.stack(out_list, axis=-1)
    ws = (gout_all * p['a']).sum(1, keepdims=True) + p['a_bias']
    ws = jax.nn.softmax(ws, axis=-1)
    we = ws[batch, 0, :]
    o = (out_all * we[:, None, :]).sum(-1)

    x2p = _make_sc_bucket_scatter(KPP2, E, 128, 5008, 2, False, E)(
        o, gx2.reshape(KPP2 // 128, 128), dlx2.reshape(KPP2 // 128, 128),
        poff3, z128)
    x2 = x + x2p[:, :5000].reshape(N, F)

    def bn(v, g, b):
        return g * (v - v.mean(0)) / jnp.sqrt(v.var(0) + 1e-5) + b

    def prelu(v, w):
        return jnp.where(v >= 0.0, v, w * v)

    y = bn(x2, p['bn1_g'], p['bn1_b']) @ p['l1_W'] + p['l1_b']
    hh = prelu(bn(y, p['bn2_g'], p['bn2_b']), p['pr2']) @ p['l2_W'] + p['l2_b']
    hh = prelu(bn(hh, p['bn3_g'], p['bn3_b']), p['pr3']) @ p['l3_W'] + p['l3_b']
    y = (hh + y) / 2.0
    hh = prelu(bn(y, p['bn4_g'], p['bn4_b']), p['pr4']) @ p['l4_W'] + p['l4_b']
    y = (hh + y) / 2.0
    y = prelu(bn(y, p['bn5_g'], p['bn5_b']), p['pr5']) @ p['l5_W'] + p['l5_b']
    return y


# speedup vs baseline: 1.3925x; 1.1766x over previous
import functools

import jax
import jax.numpy as jnp
from jax import lax
from jax.experimental import pallas as pl
from jax.experimental.pallas import tpu as pltpu
from jax.experimental.pallas import tpu_sc as plsc


@functools.lru_cache(maxsize=None)
def _make_sc_bucket_scatter(KPP, TR, ROWW, NLOC16, NBUCK, merged, EE, BS=128):
    NT = 16
    NR = KPP // BS
    ACCR = NLOC16 + 16
    ZR = ACCR // NT
    DR = NLOC16 // NT
    NZF = ZR // 16
    ZREM = ZR - NZF * 16
    mesh = plsc.VectorSubcoreMesh(core_axis_name="c", subcore_axis_name="s")

    scratch = [
        pltpu.VMEM_SHARED((ACCR, ROWW), jnp.float32),
        pltpu.VMEM((48,), jnp.int32),
        pltpu.VMEM((16, ROWW), jnp.float32),
        pltpu.VMEM((BS,), jnp.int32),
        pltpu.VMEM((BS,), jnp.int32),
        pltpu.VMEM((BS,), jnp.int32),
        pltpu.VMEM((BS,), jnp.int32),
        pltpu.VMEM((BS, ROWW), jnp.float32),
        pltpu.VMEM((BS, ROWW), jnp.float32),
        pltpu.SemaphoreType.DMA,
        pltpu.SemaphoreType.DMA,
        pltpu.SemaphoreType.DMA,
        pltpu.SemaphoreType.DMA,
    ]
    if merged:
        scratch += [
            pltpu.VMEM((BS,), jnp.int32),
            pltpu.VMEM((BS,), jnp.int32),
            pltpu.VMEM((BS, 16), jnp.float32),
            pltpu.VMEM((BS, 16), jnp.float32),
        ]

    def body(*refs):
        if merged:
            (table_h, gidx_h, dloc_h, poff_h, aspack_h, lg1_h, zrows_h, out_h,
             acc, poffv, zbuf, gb0, gb1, db0, db1, rb0, rb1,
             sem_g0, sem_g1, sem_s, sem_z,
             l1v0, l1v1, rA, rB) = refs
        else:
            (table_h, gidx_h, dloc_h, poff_h, zrows_h, out_h,
             acc, poffv, zbuf, gb0, gb1, db0, db1, rb0, rb1,
             sem_g0, sem_g1, sem_s, sem_z) = refs
        core = lax.axis_index("c")
        tid = lax.axis_index("s")
        pltpu.sync_copy(zrows_h, zbuf)
        pltpu.sync_copy(poff_h, poffv)
        iot = lax.iota(jnp.int32, 16)
        zero16 = iot * 0

        def stage_and_gather(r, gb, db, l1v, rA, rB, semg):
            if merged:
                pltpu.sync_copy(gidx_h.at[r], gb)
                pltpu.sync_copy(lg1_h.at[r], l1v)
                pltpu.sync_copy(dloc_h.at[r], db)
                gA = pltpu.async_copy(aspack_h.at[gb], rA, semg)
                gB = pltpu.async_copy(aspack_h.at[l1v], rB, semg)
                gA.wait()
                gB.wait()
                for c in range(BS // 16):
                    sl = pl.ds(c * 16, 16)
                    asv = plsc.load_gather(rA, [iot + c * 16, zero16])
                    adv = plsc.load_gather(rB, [iot + c * 16, zero16 + 1])
                    neg = jnp.where(asv + adv < 0.0, 1, 0)
                    gb[sl] = gb[sl] + neg * EE
                    db[sl] = jnp.minimum(db[sl] + neg, NLOC16)
            else:
                pltpu.sync_copy(gidx_h.at[r], gb)
                pltpu.sync_copy(dloc_h.at[r], db)
            return pltpu.async_copy(table_h.at[gb], rb0 if gb is gb0 else rb1,
                                    semg)

        for ci in range(NBUCK // 2):
            b = core * (NBUCK // 2) + ci
            pv = poffv[pl.ds(b, 16)]
            p0 = pv[0]
            p1 = pv[1]
            base_r = p0 // BS
            nb = (p1 - p0) // BS // NT
            my0 = base_r + tid * nb
            z0 = tid * ZR
            zd = [pltpu.async_copy(zbuf, acc.at[pl.ds(z0 + q * 16, 16)], sem_z)
                  for q in range(NZF)]
            if ZREM:
                zd.append(pltpu.async_copy(
                    zbuf.at[pl.ds(0, ZREM)],
                    acc.at[pl.ds(z0 + NZF * 16, ZREM)], sem_z))
            for d in zd:
                d.wait()
            plsc.subcore_barrier()

            def m_body(m, _):
                r0 = my0 + 2 * m
                r1 = jnp.where(2 * m + 1 < nb, r0 + 1, NR - 1)
                g0 = stage_and_gather(r0, gb0, db0,
                                      *( (l1v0, rA, rB) if merged
                                         else (None, None, None)), sem_g0)
                g1 = stage_and_gather(r1, gb1, db1,
                                      *( (l1v1, rA, rB) if merged
                                         else (None, None, None)), sem_g1)
                g0.wait()
                s0 = pltpu.async_copy(rb0, acc.at[db0], sem_s, add=True)
                g1.wait()
                s1 = pltpu.async_copy(rb1, acc.at[db1], sem_s, add=True)
                s0.wait()
                s1.wait()
                return _

            lax.fori_loop(0, (nb + 1) // 2, m_body, None)
            plsc.subcore_barrier()
            d0 = tid * DR
            pltpu.sync_copy(acc.at[pl.ds(d0, DR)],
                            out_h.at[b, pl.ds(d0, DR)])
            plsc.subcore_barrier()

    return functools.partial(
        pl.kernel, mesh=mesh,
        out_type=jax.ShapeDtypeStruct((NBUCK, NLOC16, ROWW), jnp.float32),
        compiler_params=pltpu.CompilerParams(use_tc_tiling_on_sc=False, needs_layout_passes=False),
        scratch_types=scratch)(body)


@functools.lru_cache(maxsize=None)
def _make_sc_gather(KPC, TR, ROWW):
    NT = 16
    PT = KPC // 32
    NB = PT // 128
    mesh = plsc.VectorSubcoreMesh(core_axis_name="c", subcore_axis_name="s")

    @functools.partial(
        pl.kernel, mesh=mesh,
        out_type=jax.ShapeDtypeStruct((KPC, ROWW), jnp.float32),
        compiler_params=pltpu.CompilerParams(use_tc_tiling_on_sc=False, needs_layout_passes=False),
        scratch_types=[
            pltpu.VMEM((128,), jnp.int32),
            pltpu.VMEM((128,), jnp.int32),
            pltpu.VMEM((128, ROWW), jnp.float32),
            pltpu.VMEM((128, ROWW), jnp.float32),
            pltpu.SemaphoreType.DMA,
            pltpu.SemaphoreType.DMA,
            pltpu.SemaphoreType.DMA,
        ])
    def k(table_h, gidx_h, out_h, gb0, gb1, rb0, rb1, sem0, sem1, sem_s):
        core = lax.axis_index("c")
        tid = lax.axis_index("s")
        w = core * 16 + tid
        r00 = (w * PT) // 128

        def m_body(m, _):
            r0 = r00 + 2 * m
            pltpu.sync_copy(gidx_h.at[r0], gb0)
            g0 = pltpu.async_copy(table_h.at[gb0], rb0, sem0)
            pltpu.sync_copy(gidx_h.at[r0 + 1], gb1)
            g1 = pltpu.async_copy(table_h.at[gb1], rb1, sem1)
            g0.wait()
            s0 = pltpu.async_copy(rb0, out_h.at[pl.ds(r0 * 128, 128)], sem_s)
            g1.wait()
            s1 = pltpu.async_copy(rb1, out_h.at[pl.ds((r0 + 1) * 128, 128)],
                                  sem_s)
            s0.wait()
            s1.wait()
            return _

        lax.fori_loop(0, NB // 2, m_body, None)

    return k


def _bucketize(dst, BCH, NBUCK, payloads, trash_vals, NLOC16):
    K = dst.shape[0]
    KPP = -(-(K + NBUCK * 2048 + 2048) // 4096) * 4096
    order = jnp.argsort(dst).astype(jnp.int32)
    ds_ = dst[order]
    off_raw = jnp.searchsorted(ds_, jnp.arange(NBUCK, dtype=dst.dtype) * BCH
                               ).astype(jnp.int32)
    off_raw1 = jnp.concatenate([off_raw, jnp.full((1,), K, jnp.int32)])
    cnt = jnp.diff(off_raw1)
    pc = -(-cnt // 2048) * 2048
    poff = jnp.concatenate([jnp.zeros((1,), jnp.int32),
                            jnp.cumsum(pc).astype(jnp.int32)])
    i = jnp.arange(KPP, dtype=jnp.int32)
    b_of = jnp.clip(jnp.searchsorted(poff, i, side='right') - 1,
                    0, NBUCK - 1).astype(jnp.int32)
    rel = i - poff[b_of]
    valid = rel < cnt[b_of]
    src = order[jnp.clip(off_raw[b_of] + rel, 0, K - 1)]
    outs = [jnp.where(valid, arr[src].astype(jnp.int32), tv)
            for arr, tv in zip(payloads, trash_vals)]
    dloc = jnp.where(valid, dst[src] % BCH, NLOC16).astype(jnp.int32)
    poff40 = jnp.zeros((48,), jnp.int32).at[:NBUCK + 1].set(poff)
    return outs, dloc, poff40, KPP


def _seg_sum(vals, seg, num):
    return jax.ops.segment_sum(vals, seg, num_segments=num)


def _combine_body(a_ref, b_ref, o_ref):
    o_ref[...] = a_ref[...] + b_ref[...]


def _pl_add(a, b):
    E, F = a.shape
    blk = 2000
    return pl.pallas_call(
        _combine_body,
        out_shape=jax.ShapeDtypeStruct((E, F), jnp.float32),
        grid=(E // blk,),
        in_specs=[pl.BlockSpec((blk, F), lambda i: (i, 0)),
                  pl.BlockSpec((blk, F), lambda i: (i, 0))],
        out_specs=pl.BlockSpec((blk, F), lambda i: (i, 0)),
    )(a, b)


def kernel(x, edge_index, edge_attr, line_graph_edge_index, edge_index_batch, params):
    p = params
    N, F = x.shape
    E = edge_index.shape[1]
    B = 128
    T = 3
    lg0 = line_graph_edge_index[0]
    lg1 = line_graph_edge_index[1]
    ei0, ei1 = edge_index[0], edge_index[1]
    batch = edge_index_batch

    z128 = jnp.zeros((16, 128), jnp.float32)
    z144 = jnp.zeros((16, 144), jnp.float32)

    (lg0p, lg1p), dloc5, poff33, KPP = _bucketize(
        lg1, 5000, 32, [lg0, lg1], [0, 0], 10000)
    dloc5x2 = jnp.minimum(dloc5 * 2, 10000)
    dloc10 = jnp.where(dloc5 == 10000, 10000, lg1p % 10000)
    poff17 = jnp.zeros((48,), jnp.int32).at[:17].set(poff33[0:34:2][:17])

    (gx2,), dlx2, poff3, KPP2 = _bucketize(
        ei1, 5000, 2, [jnp.arange(E, dtype=jnp.int32)], [0], 5008)

    edge_u = x @ p['Wu']
    edge_v = x @ p['Wv']
    edge_uv = edge_attr @ p['We']
    uv_tab = jnp.concatenate([edge_u, edge_v], axis=0)
    KPC = 327680
    gidx_uv = jnp.concatenate([ei0, ei1 + N]).astype(jnp.int32)
    gidx_uv = jnp.pad(gidx_uv, (0, KPC - 2 * E)).reshape(KPC // 128, 128)
    uvg = _make_sc_gather(KPC, 2 * N, F)(uv_tab, gidx_uv)
    ea = (uvg[:E] + uvg[E:2 * E] + edge_uv) / 3.0

    vs2 = p['gat_W'] @ p['gat_att_src']
    vd2 = p['gat_W'] @ p['gat_att_dst']

    gs2 = lg0p.reshape(KPP // 128, 128)
    dl10 = dloc10.reshape(KPP // 128, 128)
    gs64 = lg0p.reshape(KPP // 64, 64)
    dl5_64 = dloc5x2.reshape(KPP // 64, 64)
    lg1r64 = lg1p.reshape(KPP // 64, 64)

    ones_tab = jnp.zeros((8, 16), jnp.float32).at[0, 0].set(1.0)
    gz = jnp.zeros((KPP // 128, 128), jnp.int32)
    z16 = jnp.zeros((16, 16), jnp.float32)
    indeg = _make_sc_bucket_scatter(KPP, 8, 16, 10000, 16, False, E)(
        ones_tab, gz, dl10, poff17, z16).reshape(E, 16)[:, 0]
    dis = (indeg + 1.0) ** -0.5

    onehot = jax.nn.one_hot(batch, B, dtype=jnp.float32)


    out = ea
    out_list = []
    gout_list = []
    for _ in range(T):
        agg = _make_sc_bucket_scatter(KPP, E, 128, 10000, 16, False, E)(
            out, gs2, dl10, poff17, z128).reshape(E, F)
        out = _pl_add(ea, agg)

        h = out @ p['gat_W']
        a_s = out @ vs2
        a_d = out @ vd2
        h1 = (out @ p['att_gcn_W'])[:, 0]
        score_f = out @ p['fbtl_W'] + p['fbtl_b']
        gh1 = dis * h1

        eas = jnp.exp(a_s)
        eas2 = jnp.exp(0.2 * a_s)
        pad14 = jnp.zeros((E, 14), jnp.float32)
        TH = jnp.concatenate([
            jnp.concatenate([eas[:, None] * h, eas[:, None], gh1[:, None],
                             pad14], axis=1),
            jnp.concatenate([eas2[:, None] * h, eas2[:, None], gh1[:, None],
                             pad14], axis=1)], axis=0)
        aspack = jnp.concatenate(
            [a_s[:, None], a_d[:, None], jnp.zeros((E, 14), jnp.float32)],
            axis=1)
        gat = _make_sc_bucket_scatter(KPP, 2 * E, 144, 10000, 32, True, E,
                                      BS=64)(
            TH, gs64, dl5_64, poff33, aspack, lg1r64, z144)
        gat = gat.reshape(E, 2, 144)
        Spos = gat[:, 0, :F]
        Sneg = gat[:, 1, :F]
        cpos = gat[:, 0, F]
        cneg = gat[:, 1, F]
        s_lg = gat[:, 0, F + 1] + gat[:, 1, F + 1]

        score_s = dis * s_lg + dis * dis * h1 + p['att_gcn_b'][0]
        score = score_s[:, None] * 0.6 + score_f * 0.4

        e_self = jnp.exp(jax.nn.leaky_relu(a_s + a_d, 0.2))
        ead = jnp.exp(a_d)
        ead2 = jnp.exp(0.2 * a_d)
        z = ead * cpos + ead2 * cneg + e_self
        num = ead[:, None] * Spos + ead2[:, None] * Sneg + e_self[:, None] * h
        xf = num / (z + 1e-16)[:, None] + p['gat_b']

        es = jnp.exp(score)
        P = onehot.T @ jnp.concatenate([xf * es, es], axis=1)
        gout = P[:, :F] / (P[:, F:] + 1e-16)

        out_list.append(out)
        gout_list.append(jnp.tanh(gout @ p['lin_gout_W'] + p['lin_gout_b']))

    gout_all = jnp.stack(gout_list, axis=-1)
    out_all = jnp.stack(out_list, axis=-1)
    ws = (gout_all * p['a']).sum(1, keepdims=True) + p['a_bias']
    ws = jax.nn.softmax(ws, axis=-1)
    we = onehot @ ws[:, 0, :]
    o = (out_all * we[:, None, :]).sum(-1)

    x2p = _make_sc_bucket_scatter(KPP2, E, 128, 5008, 2, False, E)(
        o, gx2.reshape(KPP2 // 128, 128), dlx2.reshape(KPP2 // 128, 128),
        poff3, z128)
    x2 = x + x2p[:, :5000].reshape(N, F)

    onesr = jnp.ones((1, N), jnp.float32)

    def bn(v, g, b):
        m = (onesr @ v) / N
        v2 = (onesr @ (v * v)) / N
        var = v2 - m * m
        return g * (v - m) / jnp.sqrt(var + 1e-5) + b

    def prelu(v, w):
        return jnp.where(v >= 0.0, v, w * v)

    y = bn(x2, p['bn1_g'], p['bn1_b']) @ p['l1_W'] + p['l1_b']
    hh = prelu(bn(y, p['bn2_g'], p['bn2_b']), p['pr2']) @ p['l2_W'] + p['l2_b']
    hh = prelu(bn(hh, p['bn3_g'], p['bn3_b']), p['pr3']) @ p['l3_W'] + p['l3_b']
    y = (hh + y) / 2.0
    hh = prelu(bn(y, p['bn4_g'], p['bn4_b']), p['pr4']) @ p['l4_W'] + p['l4_b']
    y = (hh + y) / 2.0
    y = prelu(bn(y, p['bn5_g'], p['bn5_b']), p['pr5']) @ p['l5_W'] + p['l5_b']
    return y
